# baseline (speedup 1.0000x reference)
import functools

import jax
import jax.numpy as jnp
from jax import lax
from jax.experimental import pallas as pl
from jax.experimental.pallas import tpu as pltpu
from jax.experimental.pallas import tpu_sc as plsc

_N = 10000
_E = 320000
_D = 128
_NPAD = 10240
_ROWS_PER_TILE = _NPAD // 16

_NW = 32
_C = 128
_K = 8
_EPAD = 327680
_EROWS = _EPAD // _C
_NSUP = _EROWS // _K // _NW

_BR = 2048


def _mm_body(x_ref, w_ref, o_ref):
    o_ref[...] = jnp.dot(x_ref[...], w_ref[...],
                         preferred_element_type=jnp.float32)


def _matmul(x, w):
    return pl.pallas_call(
        _mm_body,
        grid=(_NPAD // _BR,),
        in_specs=[pl.BlockSpec((_BR, _D), lambda i: (i, 0)),
                  pl.BlockSpec((_D, _D), lambda i: (0, 0))],
        out_specs=pl.BlockSpec((_BR, _D), lambda i: (i, 0)),
        out_shape=jax.ShapeDtypeStruct((_NPAD, _D), jnp.float32),
    )(x, w)


def _norm_from_deg(d_ref):
    ones = jnp.ones((_NW, 1), jnp.float32)
    deg = lax.dot_general(d_ref[...], ones, (((0,), (0,)), ((), ())),
                          preferred_element_type=jnp.float32)
    return 1.0 / jnp.clip(deg, 1.0, None)


def _mid_body(a_ref, d_ref, b_ref, w_ref, o_ref):
    a = a_ref[0] + a_ref[1]
    h = a * _norm_from_deg(d_ref) + b_ref[...]
    h = jnp.maximum(h, 0.0)
    o_ref[...] = jnp.dot(h, w_ref[...], preferred_element_type=jnp.float32)


def _mid(agg, deg, b, w):
    return pl.pallas_call(
        _mid_body,
        grid=(_NPAD // _BR,),
        in_specs=[pl.BlockSpec((2, _BR, _D), lambda i: (0, i, 0)),
                  pl.BlockSpec((_NW, _BR), lambda i: (0, i)),
                  pl.BlockSpec((1, _D), lambda i: (0, 0)),
                  pl.BlockSpec((_D, _D), lambda i: (0, 0))],
        out_specs=pl.BlockSpec((_BR, _D), lambda i: (i, 0)),
        out_shape=jax.ShapeDtypeStruct((_NPAD, _D), jnp.float32),
    )(agg, deg, b, w)


def _fin_body(a_ref, d_ref, b_ref, o_ref):
    a = a_ref[0] + a_ref[1]
    o_ref[...] = a * _norm_from_deg(d_ref) + b_ref[...]


def _fin(agg, deg, b):
    return pl.pallas_call(
        _fin_body,
        grid=(_NPAD // _BR,),
        in_specs=[pl.BlockSpec((2, _BR, _D), lambda i: (0, i, 0)),
                  pl.BlockSpec((_NW, _BR), lambda i: (0, i)),
                  pl.BlockSpec((1, _D), lambda i: (0, 0))],
        out_specs=pl.BlockSpec((_BR, _D), lambda i: (i, 0)),
        out_shape=jax.ShapeDtypeStruct((_NPAD, _D), jnp.float32),
    )(agg, deg, b)


def _make_agg():
    mesh = plsc.VectorSubcoreMesh(core_axis_name="c", subcore_axis_name="s",
                                  num_cores=2, num_subcores=16)
    out_type = [jax.ShapeDtypeStruct((2, _NPAD, _D), jnp.float32)]
    scratch = [
        pltpu.VMEM((_K, _C), jnp.int32),
        pltpu.VMEM((_K, _C), jnp.int32),
        pltpu.VMEM((_K, _C), jnp.float32),
        pltpu.VMEM((2, _C, _D), jnp.float32),
        pltpu.VMEM_SHARED((_NPAD, _D), jnp.float32),
        pltpu.SemaphoreType.DMA,
        pltpu.SemaphoreType.DMA,
    ]

    def body(h_hbm, src_hbm, dst_hbm, w_hbm, z128_hbm,
             agg_out, src_v, dst_v, w_v, rows_v, agg_sh, sem, ssem):
        cid = lax.axis_index("c")
        sid = lax.axis_index("s")
        wid = sid * 2 + cid
        rbase = sid * _ROWS_PER_TILE

        stage_v = rows_v.at[0]
        pltpu.sync_copy(z128_hbm, stage_v)
        for i in range(_ROWS_PER_TILE // 128):
            pltpu.sync_copy(stage_v, agg_sh.at[pl.ds(rbase + i * 128, 128)])
        plsc.subcore_barrier()

        def scale_chunk(buf, j):
            def scale_group(g, carry):
                wv = w_v[j, pl.ds(g * 16, 16)]
                for t in range(16):
                    ws = jnp.full((16,), wv[t])
                    e = g * 16 + t
                    for k in range(_D // 16):
                        sl = pl.ds(k * 16, 16)
                        buf[e, sl] = buf[e, sl] * ws
                return carry
            lax.fori_loop(0, _C // 16, scale_group, 0, unroll=2)

        def _drain_scatter():
            pltpu.make_async_copy(
                rows_v.at[0], agg_sh.at[pl.ds(0, _C)], ssem).wait()

        def super_body(s, carry):
            erow = (wid + s * _NW) * _K
            pltpu.sync_copy(src_hbm.at[pl.ds(erow, _K)], src_v)
            pltpu.sync_copy(dst_hbm.at[pl.ds(erow, _K)], dst_v)
            pltpu.sync_copy(w_hbm.at[pl.ds(erow, _K)], w_v)
            cps = [pltpu.async_copy(h_hbm.at[src_v.at[0]], rows_v.at[0], sem)]
            for j in range(_K):
                if j >= 1:
                    _drain_scatter()
                if j + 1 < _K:
                    cps.append(pltpu.async_copy(
                        h_hbm.at[src_v.at[j + 1]], rows_v.at[(j + 1) % 2],
                        sem))
                cps[j].wait()
                buf = rows_v.at[j % 2]
                scale_chunk(buf, j)
                pltpu.async_copy(buf, agg_sh.at[dst_v.at[j]], ssem, add=True)
            _drain_scatter()
            return carry

        lax.fori_loop(0, _NSUP, super_body, 0)
        plsc.subcore_barrier()

        for i in range(_ROWS_PER_TILE // 128):
            sl = pl.ds(rbase + i * 128, 128)
            pltpu.sync_copy(agg_sh.at[sl], stage_v)
            pltpu.sync_copy(stage_v, agg_out.at[cid, sl])

    return pl.kernel(body, out_type=out_type, mesh=mesh,
                     scratch_types=scratch)


def _make_deg():
    mesh = plsc.VectorSubcoreMesh(core_axis_name="c", subcore_axis_name="s",
                                  num_cores=2, num_subcores=16)
    out_type = [jax.ShapeDtypeStruct((_NW, _NPAD), jnp.float32)]
    scratch = [
        pltpu.VMEM((_K, _C), jnp.int32),
        pltpu.VMEM((_NPAD,), jnp.float32),
    ]

    def body(dst_hbm, z1_hbm, deg_out, dst_v, hist_v):
        cid = lax.axis_index("c")
        sid = lax.axis_index("s")
        wid = sid * 2 + cid
        ones = jnp.full((16,), 1.0, jnp.float32)

        pltpu.sync_copy(z1_hbm, hist_v)

        def super_body(s, carry):
            erow = (wid + s * _NW) * _K
            pltpu.sync_copy(dst_hbm.at[pl.ds(erow, _K)], dst_v)
            for j in range(_K):
                def grp(g, c):
                    idx = dst_v[j, pl.ds(g * 16, 16)]
                    plsc.addupdate_scatter(hist_v, [idx], ones)
                    return c
                lax.fori_loop(0, _C // 16, grp, 0)
            return carry

        lax.fori_loop(0, _NSUP, super_body, 0)
        pltpu.sync_copy(hist_v, deg_out.at[wid])

    return pl.kernel(body, out_type=out_type, mesh=mesh,
                     scratch_types=scratch,
                     compiler_params=pltpu.CompilerParams(
                         needs_layout_passes=False))


_agg_edges = _make_agg()
_deg_edges = _make_deg()


@jax.jit
def kernel(x, edge_index, edge_weight, W1, b1, W2, b2):
    xp = jnp.pad(x, ((0, _NPAD - _N), (0, 0)))
    npad_e = _EPAD - _E
    src = jnp.pad(edge_index[0], (0, npad_e)).reshape(_EROWS, _C)
    dst = jnp.pad(edge_index[1], (0, npad_e),
                  constant_values=_N).reshape(_EROWS, _C)
    ew = jnp.pad(edge_weight, (0, npad_e)).reshape(_EROWS, _C)
    z128 = jnp.zeros((128, _D), jnp.float32)
    z1 = jnp.zeros((_NPAD,), jnp.float32)
    b1r = b1.reshape(1, _D)
    b2r = b2.reshape(1, _D)

    h1 = _matmul(xp, W1)
    (deg,) = _deg_edges(dst, z1)
    (agg1,) = _agg_edges(h1, src, dst, ew, z128)
    h2 = _mid(agg1, deg, b1r, W2)
    (agg2,) = _agg_edges(h2, src, dst, ew, z128)
    out = _fin(agg2, deg, b2r)
    return out[:_N]


# speedup vs baseline: 1.0129x; 1.0129x over previous
import functools

import jax
import jax.numpy as jnp
from jax import lax
from jax.experimental import pallas as pl
from jax.experimental.pallas import tpu as pltpu
from jax.experimental.pallas import tpu_sc as plsc

_N = 10000
_E = 320000
_D = 128
_NPAD = 10240
_ROWS_PER_TILE = _NPAD // 16

_NW = 32
_C = 128
_K = 8
_EPAD = 327680
_EROWS = _EPAD // _C
_NSUP = _EROWS // _K // _NW

_BR = 2048


def _mm_body(x_ref, w_ref, o_ref):
    o_ref[...] = jnp.dot(x_ref[...], w_ref[...],
                         preferred_element_type=jnp.float32)


def _matmul(x, w):
    return pl.pallas_call(
        _mm_body,
        grid=(_NPAD // _BR,),
        in_specs=[pl.BlockSpec((_BR, _D), lambda i: (i, 0)),
                  pl.BlockSpec((_D, _D), lambda i: (0, 0))],
        out_specs=pl.BlockSpec((_BR, _D), lambda i: (i, 0)),
        out_shape=jax.ShapeDtypeStruct((_NPAD, _D), jnp.float32),
    )(x, w)


def _norm_from_deg(d_ref):
    ones = jnp.ones((_NW, 1), jnp.float32)
    deg = lax.dot_general(d_ref[...], ones, (((0,), (0,)), ((), ())),
                          preferred_element_type=jnp.float32)
    return 1.0 / jnp.clip(deg, 1.0, None)


def _mid_body(a_ref, d_ref, b_ref, w_ref, o_ref):
    a = a_ref[0] + a_ref[1]
    h = a * _norm_from_deg(d_ref) + b_ref[...]
    h = jnp.maximum(h, 0.0)
    o_ref[...] = jnp.dot(h, w_ref[...], preferred_element_type=jnp.float32)


def _mid(agg, deg, b, w):
    return pl.pallas_call(
        _mid_body,
        grid=(_NPAD // _BR,),
        in_specs=[pl.BlockSpec((2, _BR, _D), lambda i: (0, i, 0)),
                  pl.BlockSpec((_NW, _BR), lambda i: (0, i)),
                  pl.BlockSpec((1, _D), lambda i: (0, 0)),
                  pl.BlockSpec((_D, _D), lambda i: (0, 0))],
        out_specs=pl.BlockSpec((_BR, _D), lambda i: (i, 0)),
        out_shape=jax.ShapeDtypeStruct((_NPAD, _D), jnp.float32),
    )(agg, deg, b, w)


def _fin_body(a_ref, d_ref, b_ref, o_ref):
    a = a_ref[0] + a_ref[1]
    o_ref[...] = a * _norm_from_deg(d_ref) + b_ref[...]


def _fin(agg, deg, b):
    return pl.pallas_call(
        _fin_body,
        grid=(_NPAD // _BR,),
        in_specs=[pl.BlockSpec((2, _BR, _D), lambda i: (0, i, 0)),
                  pl.BlockSpec((_NW, _BR), lambda i: (0, i)),
                  pl.BlockSpec((1, _D), lambda i: (0, 0))],
        out_specs=pl.BlockSpec((_BR, _D), lambda i: (i, 0)),
        out_shape=jax.ShapeDtypeStruct((_NPAD, _D), jnp.float32),
    )(agg, deg, b)


def _make_agg():
    mesh = plsc.VectorSubcoreMesh(core_axis_name="c", subcore_axis_name="s",
                                  num_cores=2, num_subcores=16)
    out_type = [jax.ShapeDtypeStruct((2, _NPAD, _D), jnp.float32)]
    scratch = [
        pltpu.VMEM((_K, _C), jnp.int32),
        pltpu.VMEM((_K, _C), jnp.int32),
        pltpu.VMEM((_K, _C), jnp.float32),
        pltpu.VMEM((2, _C, _D), jnp.float32),
        pltpu.VMEM_SHARED((_NPAD, _D), jnp.float32),
        pltpu.SemaphoreType.DMA,
    ]

    def body(h_hbm, src_hbm, dst_hbm, w_hbm, z128_hbm,
             agg_out, src_v, dst_v, w_v, rows_v, agg_sh, sem):
        cid = lax.axis_index("c")
        sid = lax.axis_index("s")
        wid = sid * 2 + cid
        rbase = sid * _ROWS_PER_TILE

        stage_v = rows_v.at[0]
        pltpu.sync_copy(z128_hbm, stage_v)
        for i in range(_ROWS_PER_TILE // 128):
            pltpu.sync_copy(stage_v, agg_sh.at[pl.ds(rbase + i * 128, 128)])
        plsc.subcore_barrier()

        def scale_chunk(buf, j):
            def scale_group(g, carry):
                wv = w_v[j, pl.ds(g * 16, 16)]
                for t in range(16):
                    ws = jnp.full((16,), wv[t])
                    e = g * 16 + t
                    for k in range(_D // 16):
                        sl = pl.ds(k * 16, 16)
                        buf[e, sl] = buf[e, sl] * ws
                return carry
            lax.fori_loop(0, _C // 16, scale_group, 0)

        def super_body(s, carry):
            erow = (wid + s * _NW) * _K
            pltpu.sync_copy(src_hbm.at[pl.ds(erow, _K)], src_v)
            pltpu.sync_copy(dst_hbm.at[pl.ds(erow, _K)], dst_v)
            pltpu.sync_copy(w_hbm.at[pl.ds(erow, _K)], w_v)
            cps = [pltpu.async_copy(h_hbm.at[src_v.at[0]], rows_v.at[0], sem)]
            for j in range(_K):
                if j + 1 < _K:
                    cps.append(pltpu.async_copy(
                        h_hbm.at[src_v.at[j + 1]], rows_v.at[(j + 1) % 2],
                        sem))
                cps[j].wait()
                buf = rows_v.at[j % 2]
                scale_chunk(buf, j)
                pltpu.sync_copy(buf, agg_sh.at[dst_v.at[j]], add=True)
            return carry

        lax.fori_loop(0, _NSUP, super_body, 0)
        plsc.subcore_barrier()

        for i in range(_ROWS_PER_TILE // 128):
            sl = pl.ds(rbase + i * 128, 128)
            pltpu.sync_copy(agg_sh.at[sl], stage_v)
            pltpu.sync_copy(stage_v, agg_out.at[cid, sl])

    return pl.kernel(body, out_type=out_type, mesh=mesh,
                     scratch_types=scratch)


def _make_deg():
    mesh = plsc.VectorSubcoreMesh(core_axis_name="c", subcore_axis_name="s",
                                  num_cores=2, num_subcores=16)
    out_type = [jax.ShapeDtypeStruct((_NW, _NPAD), jnp.float32)]
    scratch = [
        pltpu.VMEM((_K, _C), jnp.int32),
        pltpu.VMEM((_NPAD,), jnp.float32),
    ]

    def body(dst_hbm, z1_hbm, deg_out, dst_v, hist_v):
        cid = lax.axis_index("c")
        sid = lax.axis_index("s")
        wid = sid * 2 + cid
        ones = jnp.full((16,), 1.0, jnp.float32)

        pltpu.sync_copy(z1_hbm, hist_v)

        def super_body(s, carry):
            erow = (wid + s * _NW) * _K
            pltpu.sync_copy(dst_hbm.at[pl.ds(erow, _K)], dst_v)
            for j in range(_K):
                def grp(g, c):
                    idx = dst_v[j, pl.ds(g * 16, 16)]
                    plsc.addupdate_scatter(hist_v, [idx], ones)
                    return c
                lax.fori_loop(0, _C // 16, grp, 0)
            return carry

        lax.fori_loop(0, _NSUP, super_body, 0)
        pltpu.sync_copy(hist_v, deg_out.at[wid])

    return pl.kernel(body, out_type=out_type, mesh=mesh,
                     scratch_types=scratch,
                     compiler_params=pltpu.CompilerParams(
                         needs_layout_passes=False))


_agg_edges = _make_agg()
_deg_edges = _make_deg()


@jax.jit
def kernel(x, edge_index, edge_weight, W1, b1, W2, b2):
    xp = jnp.pad(x, ((0, _NPAD - _N), (0, 0)))
    npad_e = _EPAD - _E
    src = jnp.pad(edge_index[0], (0, npad_e)).reshape(_EROWS, _C)
    dst = jnp.pad(edge_index[1], (0, npad_e),
                  constant_values=_N).reshape(_EROWS, _C)
    ew = jnp.pad(edge_weight, (0, npad_e)).reshape(_EROWS, _C)
    z128 = jnp.zeros((128, _D), jnp.float32)
    z1 = jnp.zeros((_NPAD,), jnp.float32)
    b1r = b1.reshape(1, _D)
    b2r = b2.reshape(1, _D)

    h1 = _matmul(xp, W1)
    (deg,) = _deg_edges(dst, z1)
    (agg1,) = _agg_edges(h1, src, dst, ew, z128)
    h2 = _mid(agg1, deg, b1r, W2)
    (agg2,) = _agg_edges(h2, src, dst, ew, z128)
    out = _fin(agg2, deg, b2r)
    return out[:_N]
